# baseline (speedup 1.0000x reference)
import math

import jax
import jax.numpy as jnp
from jax import lax
from jax.experimental import pallas as pl
from jax.experimental.pallas import tpu as pltpu
from jax.experimental.pallas import tpu_sc as plsc

_B, _S, _D = 4, 2048, 1024
_SCALE = math.sqrt(_D)
_NW = 32
_PPW = _S // _NW
_RPW = _B * _PPW
_CH = 16
_NCH = _RPW // _CH
_QPB = _PPW // _CH
_LANES = 16
_VPR = _D // _LANES
_NBUF = 3


def _embed_kernel(x_hbm, tok_hbm, pos_hbm, out_hbm, idx_v, pos_v,
                  tok0, tok1, tok2, gs0, gs1, gs2, ss0, ss1, ss2, psem, isem):
    toks = (tok0, tok1, tok2)
    gsems = (gs0, gs1, gs2)
    ssems = (ss0, ss1, ss2)

    c = lax.axis_index("c")
    s = lax.axis_index("s")
    wid = s * 2 + c
    pos0 = wid * _PPW

    pload = pltpu.async_copy(pos_hbm.at[pl.ds(pos0, _PPW)], pos_v, psem)

    icopies = [pltpu.async_copy(x_hbm.at[pl.ds(b * _S + pos0, _PPW)],
                                idx_v.at[b], isem)
               for b in range(_B)]
    for cp in icopies:
        cp.wait()

    def start_gather(ch):
        b, q = ch // _QPB, ch % _QPB
        return pltpu.async_copy(
            tok_hbm.at[idx_v.at[b, pl.ds(q * _CH, _CH)]],
            toks[ch % _NBUF], gsems[ch % _NBUF])

    loads = [None] * _NCH
    stores = [None] * _NCH
    loads[0] = start_gather(0)
    pload.wait()
    for ch in range(_NCH):
        b, q = ch // _QPB, ch % _QPB
        buf = ch % _NBUF
        if ch + 1 < _NCH:
            if ch >= 2 and stores[ch - 2] is not None:
                stores[ch - 2].wait()
            loads[ch + 1] = start_gather(ch + 1)
        loads[ch].wait()

        q16 = q * _CH

        def row_body(r, carry):
            for k in range(_VPR):
                t = toks[buf][r, pl.ds(k * _LANES, _LANES)]
                pv = pos_v[q16 + r, pl.ds(k * _LANES, _LANES)]
                toks[buf][r, pl.ds(k * _LANES, _LANES)] = t * _SCALE + pv
            return carry
        lax.fori_loop(0, _CH, row_body, 0, unroll=False)

        out_base = b * _S + pos0 + q16
        stores[ch] = pltpu.async_copy(
            toks[buf], out_hbm.at[pl.ds(out_base, _CH)], ssems[buf])
    stores[_NCH - 2].wait()
    stores[_NCH - 1].wait()


def kernel(x, token_table, pos_table):
    xf = x.reshape(_B * _S).astype(jnp.int32)
    mesh = plsc.VectorSubcoreMesh(core_axis_name="c", subcore_axis_name="s")
    run = pl.kernel(
        _embed_kernel,
        out_type=jax.ShapeDtypeStruct((_B * _S, _D), jnp.float32),
        mesh=mesh,
        scratch_types=(
            [pltpu.VMEM((_B, _PPW), jnp.int32),
             pltpu.VMEM((_PPW, _D), jnp.float32)]
            + [pltpu.VMEM((_CH, _D), jnp.float32) for _ in range(_NBUF)]
            + [pltpu.SemaphoreType.DMA for _ in range(2 * _NBUF + 2)]
        ),
    )
    out = run(xf, token_table, pos_table)
    return out.reshape(_B, _S, _D)


# speedup vs baseline: 1.1277x; 1.1277x over previous
import math

import jax
import jax.numpy as jnp
from jax import lax
from jax.experimental import pallas as pl
from jax.experimental.pallas import tpu as pltpu
from jax.experimental.pallas import tpu_sc as plsc

_B, _S, _D = 4, 2048, 1024
_SCALE = math.sqrt(_D)
_NW = 32
_RPW = (_B * _S) // _NW
_CH = 16
_NCH = _RPW // _CH
_LANES = 16
_VPR = _D // _LANES
_NTOK = 4
_NPOS = 2


def _embed_kernel(x_hbm, tok_hbm, pos_hbm, out_hbm, idx_v,
                  tok0, tok1, tok2, tok3, pos0, pos1,
                  gs0, gs1, gs2, gs3, ps0, ps1, ss0, ss1, ss2, ss3):
    toks = (tok0, tok1, tok2, tok3)
    poss = (pos0, pos1)
    gsems = (gs0, gs1, gs2, gs3)
    psems = (ps0, ps1)
    ssems = (ss0, ss1, ss2, ss3)

    c = lax.axis_index("c")
    s = lax.axis_index("s")
    wid = s * 2 + c
    base = wid * _RPW
    pos_base = lax.rem(base, _S)

    pltpu.sync_copy(x_hbm.at[pl.ds(base, _RPW)], idx_v)

    def start_gather(ch):
        b = ch % _NTOK
        return pltpu.async_copy(
            tok_hbm.at[idx_v.at[pl.ds(ch * _CH, _CH)]], toks[b], gsems[b])

    def start_pos(ch):
        b = ch % _NPOS
        return pltpu.async_copy(
            pos_hbm.at[pl.ds(pos_base + ch * _CH, _CH)], poss[b], psems[b])

    gathers = [None] * _NCH
    ploads = [None] * _NCH
    stores = [None] * _NCH
    gathers[0] = start_gather(0)
    gathers[1] = start_gather(1)
    ploads[0] = start_pos(0)
    for ch in range(_NCH):
        tb = ch % _NTOK
        pb = ch % _NPOS
        if ch + 2 < _NCH:
            if ch >= 2 and stores[ch - 2] is not None:
                stores[ch - 2].wait()
            gathers[ch + 2] = start_gather(ch + 2)
        if ch + 1 < _NCH:
            ploads[ch + 1] = start_pos(ch + 1)
        gathers[ch].wait()
        ploads[ch].wait()

        def row_body(r, carry):
            for k in range(_VPR):
                t = toks[tb][r, pl.ds(k * _LANES, _LANES)]
                pv = poss[pb][r, pl.ds(k * _LANES, _LANES)]
                toks[tb][r, pl.ds(k * _LANES, _LANES)] = t * _SCALE + pv
            return carry
        lax.fori_loop(0, _CH, row_body, 0, unroll=False)

        stores[ch] = pltpu.async_copy(
            toks[tb], out_hbm.at[pl.ds(base + ch * _CH, _CH)], ssems[tb])
    stores[_NCH - 2].wait()
    stores[_NCH - 1].wait()


def kernel(x, token_table, pos_table):
    xf = x.reshape(_B * _S).astype(jnp.int32)
    mesh = plsc.VectorSubcoreMesh(core_axis_name="c", subcore_axis_name="s")
    run = pl.kernel(
        _embed_kernel,
        out_type=jax.ShapeDtypeStruct((_B * _S, _D), jnp.float32),
        mesh=mesh,
        scratch_types=(
            [pltpu.VMEM((_RPW,), jnp.int32)]
            + [pltpu.VMEM((_CH, _D), jnp.float32)
               for _ in range(_NTOK + _NPOS)]
            + [pltpu.SemaphoreType.DMA for _ in range(2 * _NTOK + _NPOS)]
        ),
    )
    out = run(xf, token_table, pos_table)
    return out.reshape(_B, _S, _D)
